# baseline (speedup 1.0000x reference)
import functools

import jax
import jax.numpy as jnp
from jax import lax
from jax.experimental import pallas as pl
from jax.experimental.pallas import tpu as pltpu
from jax.experimental.pallas import tpu_sc as plsc

_EMB = 32
_NI = 10
_LANES = 16


def _sc_body(user_table, item_table, idx_hbm, out_hbm,
             idx_v, urows_v, irows_v, out_v, sem):
    c = lax.axis_index("c")
    s = lax.axis_index("s")

    is_worker = jnp.logical_and(c == 0, s == 0)

    @pl.when(is_worker)
    def _():
        pltpu.sync_copy(idx_hbm, idx_v)
        cp_items = pltpu.async_copy(
            item_table.at[idx_v.at[pl.ds(0, _LANES)]], irows_v, sem)
        cp_user = pltpu.async_copy(
            user_table.at[idx_v.at[pl.ds(_LANES, 8)]], urows_v, sem)
        cp_items.wait()
        cp_user.wait()

    lane = lax.iota(jnp.int32, _LANES)
    u_lo = urows_v[0, pl.ds(0, _LANES)]
    u_hi = urows_v[0, pl.ds(_LANES, _LANES)]
    acc = jnp.zeros((_LANES,), jnp.float32)
    for i in range(_NI):
        prod = (irows_v[i, pl.ds(0, _LANES)] * u_lo
                + irows_v[i, pl.ds(_LANES, _LANES)] * u_hi)
        acc = jnp.where(lane == i, jnp.sum(prod), acc)

    out_v[...] = 1.0 / (1.0 + jnp.exp(-acc))

    @pl.when(is_worker)
    def _():
        pltpu.sync_copy(out_v, out_hbm)


def kernel(user_table, item_table, user_features, item_features):
    item_idx = item_features.astype(jnp.int32)
    user_idx = user_features.astype(jnp.int32)
    pad_i = jnp.zeros((_LANES - _NI,), jnp.int32)
    pad_u = jnp.zeros((7,), jnp.int32)
    idx = jnp.concatenate([item_idx, pad_i, user_idx, pad_u])

    mesh = plsc.VectorSubcoreMesh(core_axis_name="c", subcore_axis_name="s")
    run = pl.kernel(
        _sc_body,
        out_type=jax.ShapeDtypeStruct((_LANES,), jnp.float32),
        mesh=mesh,
        compiler_params=pltpu.CompilerParams(
            needs_layout_passes=False, use_tc_tiling_on_sc=False),
        scratch_types=[
            pltpu.VMEM((24,), jnp.int32),
            pltpu.VMEM((8, _EMB), jnp.float32),
            pltpu.VMEM((_LANES, _EMB), jnp.float32),
            pltpu.VMEM((_LANES,), jnp.float32),
            pltpu.SemaphoreType.DMA,
        ],
    )
    out = run(user_table, item_table, idx)
    return out[:_NI].reshape(_NI, 1)


# speedup vs baseline: 1.0514x; 1.0514x over previous
import functools

import jax
import jax.numpy as jnp
from jax import lax
from jax.experimental import pallas as pl
from jax.experimental.pallas import tpu as pltpu
from jax.experimental.pallas import tpu_sc as plsc

_EMB = 32
_NI = 10
_LANES = 16


def _sc_body(user_table, item_table, idx_hbm, out_hbm,
             idx_v, urows_v, irows_v, out_v, sem):
    c = lax.axis_index("c")
    s = lax.axis_index("s")

    is_worker = jnp.logical_and(c == 0, s == 0)

    @pl.when(is_worker)
    def _():
        pltpu.sync_copy(idx_hbm, idx_v)
        cp_items = pltpu.async_copy(
            item_table.at[idx_v.at[pl.ds(0, _LANES)]], irows_v, sem)
        cp_user = pltpu.async_copy(
            user_table.at[idx_v.at[pl.ds(_LANES, 8)]], urows_v, sem)
        cp_items.wait()
        cp_user.wait()

    lane = lax.iota(jnp.int32, _LANES)
    u_lo = urows_v[0, pl.ds(0, _LANES)]
    u_hi = urows_v[0, pl.ds(_LANES, _LANES)]
    acc = jnp.zeros((_LANES,), jnp.float32)
    for i in range(_NI):
        prod = (irows_v[i, pl.ds(0, _LANES)] * u_lo
                + irows_v[i, pl.ds(_LANES, _LANES)] * u_hi)
        acc = jnp.where(lane == i, jnp.sum(prod), acc)

    out_v[...] = 1.0 / (1.0 + jnp.exp(-acc))

    @pl.when(is_worker)
    def _():
        pltpu.sync_copy(out_v, out_hbm)


def kernel(user_table, item_table, user_features, item_features):
    item_idx = item_features.astype(jnp.int32)
    user_idx = user_features.astype(jnp.int32)
    pad_i = jnp.zeros((_LANES - _NI,), jnp.int32)
    pad_u = jnp.zeros((7,), jnp.int32)
    idx = jnp.concatenate([item_idx, pad_i, user_idx, pad_u])

    mesh = plsc.VectorSubcoreMesh(
        core_axis_name="c", subcore_axis_name="s", num_cores=1, num_subcores=1)
    run = pl.kernel(
        _sc_body,
        out_type=jax.ShapeDtypeStruct((_LANES,), jnp.float32),
        mesh=mesh,
        compiler_params=pltpu.CompilerParams(
            needs_layout_passes=False, use_tc_tiling_on_sc=False),
        scratch_types=[
            pltpu.VMEM((24,), jnp.int32),
            pltpu.VMEM((8, _EMB), jnp.float32),
            pltpu.VMEM((_LANES, _EMB), jnp.float32),
            pltpu.VMEM((_LANES,), jnp.float32),
            pltpu.SemaphoreType.DMA,
        ],
    )
    out = run(user_table, item_table, idx)
    return out[:_NI].reshape(_NI, 1)


# speedup vs baseline: 1.2246x; 1.1648x over previous
import jax
import jax.numpy as jnp
from jax import lax
from jax.experimental import pallas as pl
from jax.experimental.pallas import tpu as pltpu
from jax.experimental.pallas import tpu_sc as plsc

_EMB = 32
_NI = 10
_LANES = 16


def _sc_body(user_table, item_table, uidx_hbm, iidx_hbm, out_hbm,
             uidx_v, iidx_v, urows_v, irows_v, out_v, sem):
    cp_ui = pltpu.async_copy(uidx_hbm, uidx_v, sem)
    cp_ii = pltpu.async_copy(iidx_hbm, iidx_v, sem)
    cp_ui.wait()
    cp_ii.wait()
    cp_user = pltpu.async_copy(user_table.at[uidx_v], urows_v, sem)
    cp_items = pltpu.async_copy(item_table.at[iidx_v], irows_v, sem)
    cp_user.wait()
    cp_items.wait()

    lane = lax.iota(jnp.int32, _LANES)
    u_lo = urows_v[0, pl.ds(0, _LANES)]
    u_hi = urows_v[0, pl.ds(_LANES, _LANES)]
    acc = jnp.zeros((_LANES,), jnp.float32)
    for i in range(_NI):
        prod = (irows_v[i, pl.ds(0, _LANES)] * u_lo
                + irows_v[i, pl.ds(_LANES, _LANES)] * u_hi)
        acc = jnp.where(lane == i, jnp.sum(prod), acc)

    out_v[...] = 1.0 / (1.0 + jnp.exp(-acc))
    pltpu.sync_copy(out_v.at[pl.ds(0, _NI)], out_hbm)


def kernel(user_table, item_table, user_features, item_features):
    mesh = plsc.VectorSubcoreMesh(
        core_axis_name="c", subcore_axis_name="s", num_cores=1, num_subcores=1)
    run = pl.kernel(
        _sc_body,
        out_type=jax.ShapeDtypeStruct((_NI,), jnp.float32),
        mesh=mesh,
        compiler_params=pltpu.CompilerParams(
            needs_layout_passes=False, use_tc_tiling_on_sc=False),
        scratch_types=[
            pltpu.VMEM((1,), jnp.int32),
            pltpu.VMEM((_NI,), jnp.int32),
            pltpu.VMEM((1, _EMB), jnp.float32),
            pltpu.VMEM((_NI, _EMB), jnp.float32),
            pltpu.VMEM((_LANES,), jnp.float32),
            pltpu.SemaphoreType.DMA,
        ],
    )
    out = run(user_table, item_table,
              user_features.astype(jnp.int32), item_features.astype(jnp.int32))
    return out.reshape(_NI, 1)
